# baseline (speedup 1.0000x reference)
import functools

import jax
import jax.numpy as jnp
from jax import lax
from jax.experimental import pallas as pl
from jax.experimental.pallas import tpu as pltpu
from jax.experimental.pallas import tpu_sc as plsc

_CH = 128
_NK = 4
_NB = 3
_PCH = 64
_PNK = 8
_PNB = 2


def kernel(queries, query_positions, query_batch_offsets, background_queries):
    n, d = queries.shape
    p = query_positions.shape[-1]
    b = query_batch_offsets.shape[0] - 1
    l = n // b
    bg2d = background_queries.reshape(b, d)
    zrow = jnp.zeros((1, p), query_positions.dtype)

    mesh = plsc.VectorSubcoreMesh(core_axis_name="c", subcore_axis_name="s")

    @functools.partial(
        pl.kernel,
        out_type=[
            jax.ShapeDtypeStruct((b * (l + 1), d), queries.dtype),
            jax.ShapeDtypeStruct((b * (l + 1), p), query_positions.dtype),
            jax.ShapeDtypeStruct((b + 1,), query_batch_offsets.dtype),
        ],
        mesh=mesh,
        scratch_types=(
            [pltpu.VMEM((_CH, d), jnp.float32) for _ in range(_NB)]
            + [pltpu.VMEM((_CH,), jnp.int32) for _ in range(_NK)]
            + [pltpu.VMEM((_PCH + 9, p), jnp.float32) for _ in range(_PNB)]
            + [
                pltpu.VMEM((9, p), jnp.float32),
                pltpu.VMEM((b, d), jnp.float32),
                pltpu.VMEM((16,), jnp.int32),
                pltpu.VMEM((32,), jnp.int32),
            ]
            + [pltpu.SemaphoreType.DMA for _ in range(2 * _NB + 2 * _PNB)]
        ),
    )
    def sc_copy(q_hbm, p_hbm, bg_hbm, z_hbm, offs_hbm, qo_hbm, po_hbm, oo_hbm,
                *refs):
        bufs = refs[0:_NB]
        idxs = refs[_NB:_NB + _NK]
        pbufs = refs[_NB + _NK:_NB + _NK + _PNB]
        xp, bgv, bgi, offs_v = refs[_NB + _NK + _PNB:_NB + _NK + _PNB + 4]
        sems = refs[_NB + _NK + _PNB + 4:]
        si = sems[0:_NB]
        so = sems[_NB:2 * _NB]
        psi = sems[2 * _NB:2 * _NB + _PNB]
        pso = sems[2 * _NB + _PNB:2 * _NB + 2 * _PNB]

        wid = lax.axis_index("s") * 2 + lax.axis_index("c")
        img = wid // 2
        h = wid % 2
        u = img % 8
        src0 = img * l + h * (_NK * _CH)
        dst0 = img * (l + 1) + h * (_NK * _CH)

        def in_cp(k):
            j = k % _NB
            row = pl.multiple_of(src0 + k * _CH, 8)
            return pltpu.async_copy(q_hbm.at[pl.ds(row, _CH)], bufs[j], si[j])

        def out_cp(k):
            j = k % _NB
            c0 = dst0 + k * _CH
            idx = idxs[k]
            for t in range(_CH // 16):
                idx[pl.ds(t * 16, 16)] = c0 + t * 16 + lax.iota(jnp.int32, 16)
            return pltpu.async_copy(bufs[j], qo_hbm.at[idx], so[j])

        h_in = [None] * _NK
        h_out = [None] * _NK
        for k in range(_NB):
            h_in[k] = in_cp(k)
        for k in range(_NK):
            h_in[k].wait()
            h_out[k] = out_cp(k)
            if k + _NB < _NK:
                h_out[k].wait()
                h_in[k + _NB] = in_cp(k + _NB)

        A = pl.multiple_of(img * (l + 1) - u, 8)
        pbase = A + h * (_PNK * _PCH)
        ps = (pbase - img) % 8
        pa0 = pl.multiple_of(pbase - img - ps, 8)

        def pin_cp(k):
            j = k % _PNB
            return pltpu.async_copy(
                p_hbm.at[pl.ds(pl.multiple_of(pa0 + k * _PCH, 8), _PCH + 8)],
                pbufs[j].at[pl.ds(1, _PCH + 8)],
                psi[j],
            )

        def pout_cp(k):
            j = k % _PNB
            return pltpu.async_copy(
                pbufs[j].at[pl.ds(1 + ps, _PCH)],
                po_hbm.at[pl.ds(pl.multiple_of(pbase + k * _PCH, 8), _PCH)],
                pso[j],
            )

        p_in = [None] * _PNK
        p_out = [None] * _PNK
        for k in range(_PNB):
            p_in[k] = pin_cp(k)
        for k in range(_PNK):
            p_in[k].wait()
            if k == 0:
                @pl.when((h == 0) & (u >= 2))
                def _():
                    pltpu.sync_copy(
                        p_hbm.at[pl.ds(pl.multiple_of(img * l - 8, 8), 8)],
                        pbufs[0].at[pl.ds(0, 8)],
                    )

                @pl.when((h == 0) & (u >= 1))
                def _():
                    pltpu.sync_copy(z_hbm, pbufs[0].at[pl.ds(8, 1)])

            p_out[k] = pout_cp(k)
            if k + _PNB < _PNK:
                p_out[k].wait()
                p_in[k + _PNB] = pin_cp(k + _PNB)

        @pl.when((h == 1) & (u == 7))
        def _():
            pltpu.sync_copy(
                p_hbm.at[pl.ds(pl.multiple_of(img * l + l - 8, 8), 8)],
                xp.at[pl.ds(0, 8)],
            )
            pltpu.sync_copy(z_hbm, xp.at[pl.ds(8, 1)])
            pltpu.sync_copy(
                xp.at[pl.ds(1, 8)],
                po_hbm.at[pl.ds(pl.multiple_of(A + _PNK * _PCH * 2, 8), 8)],
            )

        @pl.when(wid == 0)
        def _():
            pltpu.sync_copy(bg_hbm, bgv)
            bgi[pl.ds(0, 16)] = l + (l + 1) * lax.iota(jnp.int32, 16)
            pltpu.sync_copy(bgv, qo_hbm.at[bgi])

        @pl.when(wid == 1)
        def _():
            pltpu.sync_copy(offs_hbm, offs_v.at[pl.ds(0, b + 1)])
            i16 = lax.iota(jnp.int32, 16)
            offs_v[pl.ds(0, 16)] = offs_v[pl.ds(0, 16)] + i16
            offs_v[pl.ds(16, 16)] = offs_v[pl.ds(16, 16)] + i16 + 16
            pltpu.sync_copy(offs_v.at[pl.ds(0, b + 1)], oo_hbm)

        for k in range(_NK):
            if k + _NB >= _NK:
                h_out[k].wait()
        for k in range(_PNK):
            if k + _PNB >= _PNK:
                p_out[k].wait()

    qo, pos_out, new_offsets = sc_copy(
        queries, query_positions, bg2d, zrow, query_batch_offsets
    )

    is_background = jnp.zeros((b, l + 1), dtype=bool).at[:, l].set(True).reshape(-1)
    return (qo, pos_out, new_offsets, is_background)


# speedup vs baseline: 1.5975x; 1.5975x over previous
import functools

import jax
import jax.numpy as jnp
from jax import lax
from jax.experimental import pallas as pl
from jax.experimental.pallas import tpu as pltpu
from jax.experimental.pallas import tpu_sc as plsc

_CH = 64
_NK = 8
_NB = 6


def kernel(queries, query_positions, query_batch_offsets, background_queries):
    n, d = queries.shape
    p = query_positions.shape[-1]
    b = query_batch_offsets.shape[0] - 1
    l = n // b
    bg2d = background_queries.reshape(b, d)

    mesh = plsc.VectorSubcoreMesh(core_axis_name="c", subcore_axis_name="s")

    @functools.partial(
        pl.kernel,
        out_type=jax.ShapeDtypeStruct((b * (l + 1), d), queries.dtype),
        mesh=mesh,
        scratch_types=(
            [pltpu.VMEM((_CH, d), jnp.float32) for _ in range(_NB)]
            + [pltpu.VMEM((_CH,), jnp.int32) for _ in range(_NK)]
            + [
                pltpu.VMEM((b, d), jnp.float32),
                pltpu.VMEM((16,), jnp.int32),
            ]
            + [pltpu.SemaphoreType.DMA for _ in range(2 * _NB)]
        ),
    )
    def sc_copy(q_hbm, bg_hbm, qo_hbm, *refs):
        bufs = refs[0:_NB]
        idxs = refs[_NB:_NB + _NK]
        bgv, bgi = refs[_NB + _NK:_NB + _NK + 2]
        sems = refs[_NB + _NK + 2:]
        si = sems[0:_NB]
        so = sems[_NB:2 * _NB]

        wid = lax.axis_index("s") * 2 + lax.axis_index("c")
        img = wid // 2
        h = wid % 2
        src0 = img * l + h * (_NK * _CH)
        dst0 = img * (l + 1) + h * (_NK * _CH)

        def in_cp(k):
            j = k % _NB
            row = pl.multiple_of(src0 + k * _CH, 8)
            return pltpu.async_copy(q_hbm.at[pl.ds(row, _CH)], bufs[j], si[j])

        def out_cp(k):
            j = k % _NB
            c0 = dst0 + k * _CH
            idx = idxs[k]
            for t in range(_CH // 16):
                idx[pl.ds(t * 16, 16)] = c0 + t * 16 + lax.iota(jnp.int32, 16)
            return pltpu.async_copy(bufs[j], qo_hbm.at[idx], so[j])

        h_in = [None] * _NK
        h_out = [None] * _NK
        for k in range(min(_NB, _NK)):
            h_in[k] = in_cp(k)
        for k in range(_NK):
            h_in[k].wait()
            h_out[k] = out_cp(k)
            if k + _NB < _NK:
                h_out[k].wait()
                h_in[k + _NB] = in_cp(k + _NB)
        for k in range(_NK):
            if k + _NB >= _NK:
                h_out[k].wait()

        @pl.when(wid == 0)
        def _():
            pltpu.sync_copy(bg_hbm, bgv)
            bgi[pl.ds(0, 16)] = l + (l + 1) * lax.iota(jnp.int32, 16)
            pltpu.sync_copy(bgv, qo_hbm.at[bgi])

    qo = sc_copy(queries, bg2d)

    pos_out = jnp.concatenate(
        [
            query_positions.reshape(b, l, p),
            jnp.zeros((b, 1, p), query_positions.dtype),
        ],
        axis=1,
    ).reshape(b * (l + 1), p)

    new_offsets = query_batch_offsets + jnp.arange(
        b + 1, dtype=query_batch_offsets.dtype
    )
    is_background = jnp.zeros((b, l + 1), dtype=bool).at[:, l].set(True).reshape(-1)
    return (qo, pos_out, new_offsets, is_background)


# speedup vs baseline: 1.6068x; 1.0059x over previous
import functools

import jax
import jax.numpy as jnp
from jax import lax
from jax.experimental import pallas as pl
from jax.experimental.pallas import tpu as pltpu
from jax.experimental.pallas import tpu_sc as plsc

_CH = 32
_NK = 16
_NB = 8


def kernel(queries, query_positions, query_batch_offsets, background_queries):
    n, d = queries.shape
    p = query_positions.shape[-1]
    b = query_batch_offsets.shape[0] - 1
    l = n // b
    bg2d = background_queries.reshape(b, d)

    mesh = plsc.VectorSubcoreMesh(core_axis_name="c", subcore_axis_name="s")

    @functools.partial(
        pl.kernel,
        out_type=jax.ShapeDtypeStruct((b * (l + 1), d), queries.dtype),
        mesh=mesh,
        scratch_types=(
            [pltpu.VMEM((_CH, d), jnp.float32) for _ in range(_NB)]
            + [pltpu.VMEM((_CH,), jnp.int32) for _ in range(_NK)]
            + [
                pltpu.VMEM((b, d), jnp.float32),
                pltpu.VMEM((16,), jnp.int32),
            ]
            + [pltpu.SemaphoreType.DMA for _ in range(2 * _NB)]
        ),
    )
    def sc_copy(q_hbm, bg_hbm, qo_hbm, *refs):
        bufs = refs[0:_NB]
        idxs = refs[_NB:_NB + _NK]
        bgv, bgi = refs[_NB + _NK:_NB + _NK + 2]
        sems = refs[_NB + _NK + 2:]
        si = sems[0:_NB]
        so = sems[_NB:2 * _NB]

        wid = lax.axis_index("s") * 2 + lax.axis_index("c")
        img = wid // 2
        h = wid % 2
        src0 = img * l + h * (_NK * _CH)
        dst0 = img * (l + 1) + h * (_NK * _CH)

        def in_cp(k):
            j = k % _NB
            row = pl.multiple_of(src0 + k * _CH, 8)
            return pltpu.async_copy(q_hbm.at[pl.ds(row, _CH)], bufs[j], si[j])

        def out_cp(k):
            j = k % _NB
            c0 = dst0 + k * _CH
            idx = idxs[k]
            for t in range(_CH // 16):
                idx[pl.ds(t * 16, 16)] = c0 + t * 16 + lax.iota(jnp.int32, 16)
            return pltpu.async_copy(bufs[j], qo_hbm.at[idx], so[j])

        h_in = [None] * _NK
        h_out = [None] * _NK
        for k in range(min(_NB, _NK)):
            h_in[k] = in_cp(k)
        for k in range(_NK):
            h_in[k].wait()
            h_out[k] = out_cp(k)
            if k + _NB < _NK:
                h_out[k].wait()
                h_in[k + _NB] = in_cp(k + _NB)
        for k in range(_NK):
            if k + _NB >= _NK:
                h_out[k].wait()

        @pl.when(wid == 0)
        def _():
            pltpu.sync_copy(bg_hbm, bgv)
            bgi[pl.ds(0, 16)] = l + (l + 1) * lax.iota(jnp.int32, 16)
            pltpu.sync_copy(bgv, qo_hbm.at[bgi])

    qo = sc_copy(queries, bg2d)

    pos_out = jnp.concatenate(
        [
            query_positions.reshape(b, l, p),
            jnp.zeros((b, 1, p), query_positions.dtype),
        ],
        axis=1,
    ).reshape(b * (l + 1), p)

    new_offsets = query_batch_offsets + jnp.arange(
        b + 1, dtype=query_batch_offsets.dtype
    )
    is_background = jnp.zeros((b, l + 1), dtype=bool).at[:, l].set(True).reshape(-1)
    return (qo, pos_out, new_offsets, is_background)


# speedup vs baseline: 1.6134x; 1.0041x over previous
import functools

import jax
import jax.numpy as jnp
from jax import lax
from jax.experimental import pallas as pl
from jax.experimental.pallas import tpu as pltpu
from jax.experimental.pallas import tpu_sc as plsc

_CH = 64
_NK = 8
_NB = 7


def kernel(queries, query_positions, query_batch_offsets, background_queries):
    n, d = queries.shape
    p = query_positions.shape[-1]
    b = query_batch_offsets.shape[0] - 1
    l = n // b
    bg2d = background_queries.reshape(b, d)

    mesh = plsc.VectorSubcoreMesh(core_axis_name="c", subcore_axis_name="s")

    @functools.partial(
        pl.kernel,
        out_type=jax.ShapeDtypeStruct((b * (l + 1), d), queries.dtype),
        mesh=mesh,
        scratch_types=(
            [pltpu.VMEM((_CH, d), jnp.float32) for _ in range(_NB)]
            + [pltpu.VMEM((_CH,), jnp.int32) for _ in range(_NK)]
            + [
                pltpu.VMEM((b, d), jnp.float32),
                pltpu.VMEM((16,), jnp.int32),
            ]
            + [pltpu.SemaphoreType.DMA for _ in range(2 * _NB)]
        ),
    )
    def sc_copy(q_hbm, bg_hbm, qo_hbm, *refs):
        bufs = refs[0:_NB]
        idxs = refs[_NB:_NB + _NK]
        bgv, bgi = refs[_NB + _NK:_NB + _NK + 2]
        sems = refs[_NB + _NK + 2:]
        si = sems[0:_NB]
        so = sems[_NB:2 * _NB]

        wid = lax.axis_index("s") * 2 + lax.axis_index("c")
        img = wid // 2
        h = wid % 2
        src0 = img * l + h * (_NK * _CH)
        dst0 = img * (l + 1) + h * (_NK * _CH)

        def in_cp(k):
            j = k % _NB
            row = pl.multiple_of(src0 + k * _CH, 8)
            return pltpu.async_copy(q_hbm.at[pl.ds(row, _CH)], bufs[j], si[j])

        def out_cp(k):
            j = k % _NB
            c0 = dst0 + k * _CH
            idx = idxs[k]
            for t in range(_CH // 16):
                idx[pl.ds(t * 16, 16)] = c0 + t * 16 + lax.iota(jnp.int32, 16)
            return pltpu.async_copy(bufs[j], qo_hbm.at[idx], so[j])

        h_in = [None] * _NK
        h_out = [None] * _NK
        for k in range(min(_NB, _NK)):
            h_in[k] = in_cp(k)
        for k in range(_NK):
            h_in[k].wait()
            h_out[k] = out_cp(k)
            if k + _NB < _NK:
                h_out[k].wait()
                h_in[k + _NB] = in_cp(k + _NB)
        for k in range(_NK):
            if k + _NB >= _NK:
                h_out[k].wait()

        @pl.when(wid == 0)
        def _():
            pltpu.sync_copy(bg_hbm, bgv)
            bgi[pl.ds(0, 16)] = l + (l + 1) * lax.iota(jnp.int32, 16)
            pltpu.sync_copy(bgv, qo_hbm.at[bgi])

    qo = sc_copy(queries, bg2d)

    pos_out = jnp.concatenate(
        [
            query_positions.reshape(b, l, p),
            jnp.zeros((b, 1, p), query_positions.dtype),
        ],
        axis=1,
    ).reshape(b * (l + 1), p)

    new_offsets = query_batch_offsets + jnp.arange(
        b + 1, dtype=query_batch_offsets.dtype
    )
    is_background = jnp.zeros((b, l + 1), dtype=bool).at[:, l].set(True).reshape(-1)
    return (qo, pos_out, new_offsets, is_background)
